# baseline (speedup 1.0000x reference)
import functools

import jax
import jax.numpy as jnp
import numpy as np
from jax import lax
from jax.experimental import pallas as pl
from jax.experimental.pallas import tpu as pltpu
from jax.experimental.pallas import tpu_sc as plsc

B = 128
N = 100000
NC, NS, L = 2, 16, 16
NG = 16
HALFW = 49920
CW = 4992
NCH = HALFW // CW
NVC = CW // L
U = 2
XA_OFF = 2 * HALFW
XB_OFF = XA_OFF + 128
NEG = float("-inf")

_DN = lax.GatherDimensionNumbers(
    offset_dims=(), collapsed_slice_dims=(0,), start_index_map=(0,)
)


def _perm(x, perm):
    return lax.gather(
        x, perm, _DN, slice_sizes=(1,),
        mode=lax.GatherScatterMode.PROMISE_IN_BOUNDS,
    )


def _merge(m, i, m2, i2):
    tk = (m2 > m) | ((m2 == m) & (i2 < i))
    return jnp.where(tk, m2, m), jnp.where(tk, i2, i)


_mesh = plsc.VectorSubcoreMesh(
    core_axis_name="c", subcore_axis_name="s", num_cores=NC, num_subcores=NS
)


@functools.partial(
    pl.kernel,
    out_type=[
        jax.ShapeDtypeStruct((2 * NG * L,), jnp.int32),
        jax.ShapeDtypeStruct((2 * NG * L,), jnp.float32),
    ],
    mesh=_mesh,
    scratch_types=[
        pltpu.VMEM((8, CW), jnp.float32),
        pltpu.VMEM((8, CW), jnp.float32),
        pltpu.VMEM((8, 128), jnp.float32),
        pltpu.VMEM((8, 32), jnp.float32),
        pltpu.VMEM((L,), jnp.int32),
        pltpu.VMEM((L,), jnp.float32),
        pltpu.VMEM((8, L), jnp.int32),
        pltpu.VMEM((8, L), jnp.float32),
        pltpu.SemaphoreType.DMA,
        pltpu.SemaphoreType.DMA,
        pltpu.SemaphoreType.DMA,
        pltpu.SemaphoreType.DMA,
    ],
)
def _sc_argmax(x_hbm, idx_hbm, val_hbm, buf0, buf1, bufxa, bufxb,
               stg_i, stg_v, xr_i, xr_v,
               sem0, sem1, semxa, semxb):
    c = lax.axis_index("c")
    s = lax.axis_index("s")
    half = s % 2
    g = c * 8 + s // 2
    r_base = pl.multiple_of(g * 8, 8)
    col_base = half * HALFW

    bufs = (buf0, buf1)
    sems = (sem0, sem1)
    lane = lax.iota(jnp.int32, L)
    neg_v = jnp.full((L,), NEG, jnp.float32)

    def start_main(ch):
        off = pl.multiple_of(col_base + ch * CW, 128)
        return pltpu.async_copy(
            x_hbm.at[pl.ds(r_base, 8), pl.ds(off, CW)],
            bufs[ch % 2],
            sems[ch % 2],
        )

    copies = [None] * NCH
    copies[0] = start_main(0)
    copies[1] = start_main(1)
    cp_xa = pltpu.async_copy(
        x_hbm.at[pl.ds(r_base, 8), pl.ds(XA_OFF, 128)], bufxa, semxa)
    cp_xb = pltpu.async_copy(
        x_hbm.at[pl.ds(r_base, 8), pl.ds(XB_OFF, 32)], bufxb, semxb)

    prm = [neg_v for _ in range(8)]
    pri = [jnp.zeros((L,), jnp.int32) for _ in range(8)]

    for ch in range(NCH):
        copies[ch].wait()
        buf = bufs[ch % 2]
        ch_col = col_base + ch * CW
        for r8 in range(8):
            carry = []
            for u in range(U):
                carry.append(neg_v)
                carry.append(jnp.zeros((L,), jnp.int32))
                carry.append(lane + (ch_col + u * L))

            @plsc.parallel_loop(0, NVC, U, unroll=2, carry=tuple(carry))
            def scan(i, cr):
                out = []
                for u in range(U):
                    rm, ri, iv = cr[3 * u], cr[3 * u + 1], cr[3 * u + 2]
                    x = buf[r8, pl.ds((i + u) * L, L)]
                    m = x > rm
                    out.append(jnp.maximum(rm, x))
                    out.append(jnp.where(m, iv, ri))
                    out.append(iv + U * L)
                return tuple(out)

            rm, ri = scan[0], scan[1]
            for u in range(1, U):
                rm, ri = _merge(rm, ri, scan[3 * u], scan[3 * u + 1])
            prm[r8], pri[r8] = _merge(prm[r8], pri[r8], rm, ri)
        if ch + 2 < NCH:
            copies[ch + 2] = start_main(ch + 2)

    cp_xa.wait()
    cp_xb.wait()
    for r8 in range(8):
        xr_v[r8, :] = neg_v
        xr_i[r8, :] = jnp.zeros((L,), jnp.int32)

    @pl.when(half == 1)
    def _extra():
        for r8 in range(8):
            rm = neg_v
            ri = jnp.zeros((L,), jnp.int32)
            for j in range(8):
                x = bufxa[r8, pl.ds(j * L, L)]
                m = x > rm
                ri = jnp.where(m, lane + (XA_OFF + j * L), ri)
                rm = jnp.maximum(rm, x)
            for j in range(2):
                x = bufxb[r8, pl.ds(j * L, L)]
                m = x > rm
                ri = jnp.where(m, lane + (XB_OFF + j * L), ri)
                rm = jnp.maximum(rm, x)
            xr_v[r8, :] = rm
            xr_i[r8, :] = ri

    for r8 in range(8):
        prm[r8], pri[r8] = _merge(prm[r8], pri[r8], xr_v[r8, :], xr_i[r8, :])

    sv = neg_v
    si = jnp.zeros((L,), jnp.int32)
    perms = [(lane ^ sh).reshape(L, 1) for sh in (8, 4, 2, 1)]
    for r8 in range(8):
        rm, ri = prm[r8], pri[r8]
        for p in perms:
            rm, ri = _merge(rm, ri, _perm(rm, p), _perm(ri, p))
        sel = lane == r8
        sv = jnp.where(sel, rm, sv)
        si = jnp.where(sel, ri, si)

    stg_v[...] = sv
    stg_i[...] = si
    out_off = pl.multiple_of((2 * g + half) * L, 8)
    pltpu.sync_copy(stg_i, idx_hbm.at[pl.ds(out_off, L)])
    pltpu.sync_copy(stg_v, val_hbm.at[pl.ds(out_off, L)])


def _tc_merge(i0_ref, v0_ref, i1_ref, v1_ref, io_ref, vo_ref):
    v0, v1 = v0_ref[...], v1_ref[...]
    tk = v1 > v0
    io_ref[...] = jnp.where(tk, i1_ref[...], i0_ref[...])
    vo_ref[...] = jnp.where(tk, v1, v0)


def kernel(logits):
    out_idx, out_val = _sc_argmax(logits)
    pi = out_idx.reshape(NG, 2, L)
    pv = out_val.reshape(NG, 2, L)
    a_idx, ll = pl.pallas_call(
        _tc_merge,
        out_shape=[
            jax.ShapeDtypeStruct((NG, L), jnp.int32),
            jax.ShapeDtypeStruct((NG, L), jnp.float32),
        ],
    )(pi[:, 0], pv[:, 0], pi[:, 1], pv[:, 1])
    return a_idx[:, :8].reshape(B), ll[:, :8].reshape(B, 1)


# speedup vs baseline: 1.0005x; 1.0005x over previous
import functools

import jax
import jax.numpy as jnp
import numpy as np
from jax import lax
from jax.experimental import pallas as pl
from jax.experimental.pallas import tpu as pltpu
from jax.experimental.pallas import tpu_sc as plsc

B = 128
N = 100000
NC, NS, L = 2, 16, 16
NG = 16
HALFW = 49920
CW = 4992
NCH = HALFW // CW
NVC = CW // L
U = 2
XA_OFF = 2 * HALFW
XB_OFF = XA_OFF + 128
NEG = float("-inf")

_DN = lax.GatherDimensionNumbers(
    offset_dims=(), collapsed_slice_dims=(0,), start_index_map=(0,)
)


def _perm(x, perm):
    return lax.gather(
        x, perm, _DN, slice_sizes=(1,),
        mode=lax.GatherScatterMode.PROMISE_IN_BOUNDS,
    )


def _merge(m, i, m2, i2):
    tk = (m2 > m) | ((m2 == m) & (i2 < i))
    return jnp.where(tk, m2, m), jnp.where(tk, i2, i)


_mesh = plsc.VectorSubcoreMesh(
    core_axis_name="c", subcore_axis_name="s", num_cores=NC, num_subcores=NS
)


@functools.partial(
    pl.kernel,
    out_type=[
        jax.ShapeDtypeStruct((2 * NG * L,), jnp.int32),
        jax.ShapeDtypeStruct((2 * NG * L,), jnp.float32),
    ],
    mesh=_mesh,
    scratch_types=[
        pltpu.VMEM((8, CW), jnp.float32),
        pltpu.VMEM((8, CW), jnp.float32),
        pltpu.VMEM((8, 128), jnp.float32),
        pltpu.VMEM((8, 32), jnp.float32),
        pltpu.VMEM((L,), jnp.int32),
        pltpu.VMEM((L,), jnp.float32),
        pltpu.VMEM((8, L), jnp.int32),
        pltpu.VMEM((8, L), jnp.float32),
        pltpu.SemaphoreType.DMA,
        pltpu.SemaphoreType.DMA,
        pltpu.SemaphoreType.DMA,
        pltpu.SemaphoreType.DMA,
    ],
)
def _sc_argmax(x_hbm, idx_hbm, val_hbm, buf0, buf1, bufxa, bufxb,
               stg_i, stg_v, xr_i, xr_v,
               sem0, sem1, semxa, semxb):
    c = lax.axis_index("c")
    s = lax.axis_index("s")
    half = s % 2
    g = c * 8 + s // 2
    r_base = pl.multiple_of(g * 8, 8)
    col_base = half * HALFW

    bufs = (buf0, buf1)
    sems = (sem0, sem1)
    lane = lax.iota(jnp.int32, L)
    neg_v = jnp.full((L,), NEG, jnp.float32)

    def start_main(ch):
        off = pl.multiple_of(col_base + ch * CW, 128)
        return pltpu.async_copy(
            x_hbm.at[pl.ds(r_base, 8), pl.ds(off, CW)],
            bufs[ch % 2],
            sems[ch % 2],
        )

    copies = [None] * NCH
    copies[0] = start_main(0)
    copies[1] = start_main(1)
    cp_xa = pltpu.async_copy(
        x_hbm.at[pl.ds(r_base, 8), pl.ds(XA_OFF, 128)], bufxa, semxa)
    cp_xb = pltpu.async_copy(
        x_hbm.at[pl.ds(r_base, 8), pl.ds(XB_OFF, 32)], bufxb, semxb)

    prm = [neg_v for _ in range(8)]
    pri = [jnp.zeros((L,), jnp.int32) for _ in range(8)]

    for ch in range(NCH):
        copies[ch].wait()
        buf = bufs[ch % 2]
        ch_col = col_base + ch * CW
        for r8 in range(8):
            carry = []
            for u in range(U):
                carry.append(neg_v)
                carry.append(jnp.zeros((L,), jnp.int32))
                carry.append(lane + (ch_col + u * L))

            @plsc.parallel_loop(0, NVC, U, unroll=8, carry=tuple(carry))
            def scan(i, cr):
                out = []
                for u in range(U):
                    rm, ri, iv = cr[3 * u], cr[3 * u + 1], cr[3 * u + 2]
                    x = buf[r8, pl.ds((i + u) * L, L)]
                    m = x > rm
                    out.append(jnp.maximum(rm, x))
                    out.append(jnp.where(m, iv, ri))
                    out.append(iv + U * L)
                return tuple(out)

            rm, ri = scan[0], scan[1]
            for u in range(1, U):
                rm, ri = _merge(rm, ri, scan[3 * u], scan[3 * u + 1])
            prm[r8], pri[r8] = _merge(prm[r8], pri[r8], rm, ri)
        if ch + 2 < NCH:
            copies[ch + 2] = start_main(ch + 2)

    cp_xa.wait()
    cp_xb.wait()
    for r8 in range(8):
        xr_v[r8, :] = neg_v
        xr_i[r8, :] = jnp.zeros((L,), jnp.int32)

    @pl.when(half == 1)
    def _extra():
        for r8 in range(8):
            rm = neg_v
            ri = jnp.zeros((L,), jnp.int32)
            for j in range(8):
                x = bufxa[r8, pl.ds(j * L, L)]
                m = x > rm
                ri = jnp.where(m, lane + (XA_OFF + j * L), ri)
                rm = jnp.maximum(rm, x)
            for j in range(2):
                x = bufxb[r8, pl.ds(j * L, L)]
                m = x > rm
                ri = jnp.where(m, lane + (XB_OFF + j * L), ri)
                rm = jnp.maximum(rm, x)
            xr_v[r8, :] = rm
            xr_i[r8, :] = ri

    for r8 in range(8):
        prm[r8], pri[r8] = _merge(prm[r8], pri[r8], xr_v[r8, :], xr_i[r8, :])

    sv = neg_v
    si = jnp.zeros((L,), jnp.int32)
    perms = [(lane ^ sh).reshape(L, 1) for sh in (8, 4, 2, 1)]
    for r8 in range(8):
        rm, ri = prm[r8], pri[r8]
        for p in perms:
            rm, ri = _merge(rm, ri, _perm(rm, p), _perm(ri, p))
        sel = lane == r8
        sv = jnp.where(sel, rm, sv)
        si = jnp.where(sel, ri, si)

    stg_v[...] = sv
    stg_i[...] = si
    out_off = pl.multiple_of((2 * g + half) * L, 8)
    pltpu.sync_copy(stg_i, idx_hbm.at[pl.ds(out_off, L)])
    pltpu.sync_copy(stg_v, val_hbm.at[pl.ds(out_off, L)])


def _tc_merge(i0_ref, v0_ref, i1_ref, v1_ref, io_ref, vo_ref):
    v0, v1 = v0_ref[...], v1_ref[...]
    tk = v1 > v0
    io_ref[...] = jnp.where(tk, i1_ref[...], i0_ref[...])
    vo_ref[...] = jnp.where(tk, v1, v0)


def kernel(logits):
    out_idx, out_val = _sc_argmax(logits)
    pi = out_idx.reshape(NG, 2, L)
    pv = out_val.reshape(NG, 2, L)
    a_idx, ll = pl.pallas_call(
        _tc_merge,
        out_shape=[
            jax.ShapeDtypeStruct((NG, L), jnp.int32),
            jax.ShapeDtypeStruct((NG, L), jnp.float32),
        ],
    )(pi[:, 0], pv[:, 0], pi[:, 1], pv[:, 1])
    return a_idx[:, :8].reshape(B), ll[:, :8].reshape(B, 1)


# speedup vs baseline: 1.0022x; 1.0017x over previous
import functools

import jax
import jax.numpy as jnp
from jax import lax
from jax.experimental import pallas as pl
from jax.experimental.pallas import tpu as pltpu
from jax.experimental.pallas import tpu_sc as plsc

B = 128
N = 100000
NC, NS, L = 2, 16, 16
NG = 16
HALFW = 49920
CW = 4992
NCH = HALFW // CW
NVC = CW // L
U = 2
XA_OFF = 2 * HALFW
XB_OFF = XA_OFF + 128
NEG = float("-inf")

_DN = lax.GatherDimensionNumbers(
    offset_dims=(), collapsed_slice_dims=(0,), start_index_map=(0,)
)


def _perm(x, perm):
    return lax.gather(
        x, perm, _DN, slice_sizes=(1,),
        mode=lax.GatherScatterMode.PROMISE_IN_BOUNDS,
    )


def _merge(m, i, m2, i2):
    tk = (m2 > m) | ((m2 == m) & (i2 < i))
    return jnp.where(tk, m2, m), jnp.where(tk, i2, i)


_mesh = plsc.VectorSubcoreMesh(
    core_axis_name="c", subcore_axis_name="s", num_cores=NC, num_subcores=NS
)


@functools.partial(
    pl.kernel,
    out_type=[
        jax.ShapeDtypeStruct((2 * NG * L,), jnp.int32),
        jax.ShapeDtypeStruct((2 * NG * L,), jnp.float32),
    ],
    mesh=_mesh,
    scratch_types=[
        pltpu.VMEM((8, CW), jnp.float32),
        pltpu.VMEM((8, CW), jnp.float32),
        pltpu.VMEM((8, 128), jnp.float32),
        pltpu.VMEM((8, 32), jnp.float32),
        pltpu.VMEM((L,), jnp.int32),
        pltpu.VMEM((L,), jnp.float32),
        pltpu.VMEM((8, L), jnp.int32),
        pltpu.VMEM((8, L), jnp.float32),
        pltpu.SemaphoreType.DMA,
        pltpu.SemaphoreType.DMA,
        pltpu.SemaphoreType.DMA,
        pltpu.SemaphoreType.DMA,
    ],
)
def _sc_argmax(x_hbm, idx_hbm, val_hbm, buf0, buf1, bufxa, bufxb,
               stg_i, stg_v, xr_i, xr_v,
               sem0, sem1, semxa, semxb):
    c = lax.axis_index("c")
    s = lax.axis_index("s")
    half = s % 2
    g = c * 8 + s // 2
    r_base = pl.multiple_of(g * 8, 8)
    col_base = half * HALFW

    bufs = (buf0, buf1)
    sems = (sem0, sem1)
    lane = lax.iota(jnp.int32, L)
    neg_v = jnp.full((L,), NEG, jnp.float32)

    def start_main(ch):
        off = pl.multiple_of(col_base + ch * CW, 128)
        return pltpu.async_copy(
            x_hbm.at[pl.ds(r_base, 8), pl.ds(off, CW)],
            bufs[ch % 2],
            sems[ch % 2],
        )

    copies = [None] * NCH
    copies[0] = start_main(0)
    copies[1] = start_main(1)
    cp_xa = pltpu.async_copy(
        x_hbm.at[pl.ds(r_base, 8), pl.ds(XA_OFF, 128)], bufxa, semxa)
    cp_xb = pltpu.async_copy(
        x_hbm.at[pl.ds(r_base, 8), pl.ds(XB_OFF, 32)], bufxb, semxb)

    prm = [neg_v for _ in range(8)]
    pri = [jnp.zeros((L,), jnp.int32) for _ in range(8)]

    for ch in range(NCH):
        copies[ch].wait()
        buf = bufs[ch % 2]
        ch_col = col_base + ch * CW
        for r8 in range(8):
            carry = []
            for u in range(U):
                carry.append(neg_v)
                carry.append(jnp.zeros((L,), jnp.int32))
                carry.append(lane + (ch_col + u * L))

            @plsc.parallel_loop(0, NVC, U, unroll=8, carry=tuple(carry))
            def scan(i, cr):
                out = []
                for u in range(U):
                    rm, ri, iv = cr[3 * u], cr[3 * u + 1], cr[3 * u + 2]
                    x = buf[r8, pl.ds((i + u) * L, L)]
                    m = x > rm
                    out.append(jnp.maximum(rm, x))
                    out.append(jnp.where(m, iv, ri))
                    out.append(iv + U * L)
                return tuple(out)

            rm, ri = scan[0], scan[1]
            for u in range(1, U):
                rm, ri = _merge(rm, ri, scan[3 * u], scan[3 * u + 1])
            prm[r8], pri[r8] = _merge(prm[r8], pri[r8], rm, ri)
        if ch + 2 < NCH:
            copies[ch + 2] = start_main(ch + 2)

    cp_xa.wait()
    cp_xb.wait()
    for r8 in range(8):
        xr_v[r8, :] = neg_v
        xr_i[r8, :] = jnp.zeros((L,), jnp.int32)

    @pl.when(half == 1)
    def _extra():
        for r8 in range(8):
            rm = neg_v
            ri = jnp.zeros((L,), jnp.int32)
            for j in range(8):
                x = bufxa[r8, pl.ds(j * L, L)]
                m = x > rm
                ri = jnp.where(m, lane + (XA_OFF + j * L), ri)
                rm = jnp.maximum(rm, x)
            for j in range(2):
                x = bufxb[r8, pl.ds(j * L, L)]
                m = x > rm
                ri = jnp.where(m, lane + (XB_OFF + j * L), ri)
                rm = jnp.maximum(rm, x)
            xr_v[r8, :] = rm
            xr_i[r8, :] = ri

    for r8 in range(8):
        prm[r8], pri[r8] = _merge(prm[r8], pri[r8], xr_v[r8, :], xr_i[r8, :])

    sv = neg_v
    si = jnp.zeros((L,), jnp.int32)
    perms = [(lane ^ sh).reshape(L, 1) for sh in (8, 4, 2, 1)]
    for r8 in range(8):
        rm, ri = prm[r8], pri[r8]
        for p in perms:
            rm, ri = _merge(rm, ri, _perm(rm, p), _perm(ri, p))
        sel = lane == r8
        sv = jnp.where(sel, rm, sv)
        si = jnp.where(sel, ri, si)

    stg_v[...] = sv
    stg_i[...] = si
    out_off = pl.multiple_of((2 * g + half) * L, 8)
    pltpu.sync_copy(stg_i, idx_hbm.at[pl.ds(out_off, L)])
    pltpu.sync_copy(stg_v, val_hbm.at[pl.ds(out_off, L)])


def _tc_merge(i0_ref, v0_ref, i1_ref, v1_ref, io_ref, vo_ref):
    v0, v1 = v0_ref[...], v1_ref[...]
    tk = v1 > v0
    io_ref[...] = jnp.where(tk, i1_ref[...], i0_ref[...])
    vo_ref[...] = jnp.where(tk, v1, v0)


def kernel(logits):
    out_idx, out_val = _sc_argmax(logits)
    pi = out_idx.reshape(NG, 2, L)
    pv = out_val.reshape(NG, 2, L)
    a_idx, ll = pl.pallas_call(
        _tc_merge,
        out_shape=[
            jax.ShapeDtypeStruct((NG, L), jnp.int32),
            jax.ShapeDtypeStruct((NG, L), jnp.float32),
        ],
    )(pi[:, 0], pv[:, 0], pi[:, 1], pv[:, 1])
    return a_idx[:, :8].reshape(B), ll[:, :8].reshape(B, 1)
